# baseline (speedup 1.0000x reference)
import functools

import jax
import jax.numpy as jnp
from jax import lax
from jax.experimental import pallas as pl
from jax.experimental.pallas import tpu as pltpu
from jax.experimental.pallas import tpu_sc as plsc

_B = 16384
_D = 16
_NC = 2
_NS = 16
_NW = _NC * _NS
_BPW = _B // _NW


_NSEM = 16


def _gather_body(idx_hbm, tab_hbm, out_hbm, idx_v, rows_v, *sems):
    wid = lax.axis_index("s") * _NC + lax.axis_index("c")
    base = wid * _BPW
    pltpu.sync_copy(idx_hbm.at[pl.ds(base, _BPW)], idx_v)

    def fire(c):
        chunk = idx_v[pl.ds(c * 16, 16)]
        for j in range(16):
            r = chunk[j]
            pltpu.make_async_copy(
                tab_hbm.at[pl.ds(r, 1)],
                rows_v.at[pl.ds(c * 16 + j, 1)],
                sems[j % _NSEM],
            ).start()

    pl.loop(0, _BPW // 16)(fire)
    _PER_SEM = _BPW // _NSEM
    for k in range(_NSEM):
        pltpu.make_async_copy(
            tab_hbm.at[pl.ds(0, _PER_SEM)],
            rows_v.at[pl.ds(0, _PER_SEM)],
            sems[k],
        ).wait()
    pltpu.sync_copy(rows_v, out_hbm.at[pl.ds(base, _BPW)])


@jax.jit
def kernel(speaker_ids, table):
    mesh = plsc.VectorSubcoreMesh(core_axis_name="c", subcore_axis_name="s")
    fn = functools.partial(
        pl.kernel,
        mesh=mesh,
        out_type=jax.ShapeDtypeStruct((_B, _D), jnp.float32),
        scratch_types=[
            pltpu.VMEM((_BPW,), jnp.int32),
            pltpu.VMEM((_BPW, _D), jnp.float32),
        ] + [pltpu.SemaphoreType.DMA] * _NSEM,
    )(_gather_body)
    return fn(speaker_ids.astype(jnp.int32), table)


# speedup vs baseline: 3.1384x; 3.1384x over previous
import functools

import jax
import jax.numpy as jnp
from jax import lax
from jax.experimental import pallas as pl
from jax.experimental.pallas import tpu as pltpu
from jax.experimental.pallas import tpu_sc as plsc

_B = 16384
_D = 16
_NC = 2
_NS = 16
_NW = _NC * _NS
_BPW = _B // _NW
_C = 16
_NCH = _BPW // _C


def _gather_body(idx_hbm, tt_hbm, out_hbm, idx_v, blk_v, rows_v, sem):
    wid = lax.axis_index("s") * _NC + lax.axis_index("c")
    base = wid * _BPW
    pltpu.sync_copy(idx_hbm.at[pl.ds(base, _BPW)], idx_v)

    def chunk_fn(c):
        chunk = idx_v[pl.ds(c * _C, _C)]
        for k in range(_C):
            g128 = pl.multiple_of(
                lax.shift_left(lax.shift_right_logical(chunk[k], 7), 7), 128
            )
            pltpu.make_async_copy(
                tt_hbm.at[:, pl.ds(g128, 128)], blk_v.at[k], sem
            ).start()
        for k in range(_C):
            pltpu.make_async_copy(
                tt_hbm.at[pl.ds(0, _D), pl.ds(0, 128)], blk_v.at[k], sem
            ).wait()
        rr_v = lax.bitwise_and(chunk, 127)
        kk_v = lax.iota(jnp.int32, 16)
        rows16 = lax.iota(jnp.int32, 16) + (c * _C)
        for col in range(_D):
            col_v = jnp.full((16,), col, jnp.int32)
            vals = plsc.load_gather(blk_v, [kk_v, col_v, rr_v])
            plsc.store_scatter(rows_v, [rows16, col_v], vals)

    pl.loop(0, _NCH)(chunk_fn)
    pltpu.sync_copy(rows_v, out_hbm.at[pl.ds(base, _BPW)])


@jax.jit
def kernel(speaker_ids, table):
    mesh = plsc.VectorSubcoreMesh(core_axis_name="c", subcore_axis_name="s")
    fn = functools.partial(
        pl.kernel,
        mesh=mesh,
        out_type=jax.ShapeDtypeStruct((_B, _D), jnp.float32),
        scratch_types=[
            pltpu.VMEM((_BPW,), jnp.int32),
            pltpu.VMEM((_C, _D, 128), jnp.float32),
            pltpu.VMEM((_BPW, _D), jnp.float32),
            pltpu.SemaphoreType.DMA,
        ],
        compiler_params=pltpu.CompilerParams(needs_layout_passes=False),
    )(_gather_body)
    return fn(speaker_ids.astype(jnp.int32), table.T)
